# Initial kernel scaffold; baseline (speedup 1.0000x reference)
import jax
import jax.numpy as jnp
from jax.experimental import pallas as pl


def kernel(user_id, question_id, q_table, user_edge_index, user_edge_weight, question_edge_index, question_edge_weight, concept_edge_index, concept_edge_weight, student_emb, exercise_emb, knowledge_emb, W_s, b_s, W_e, b_e, W_k, b_k, W_d, b_d, W1, b1, W2, b2, W3, b3, W4, b4):
    raise NotImplementedError("write your pallas kernel here")


# speedup vs baseline: 2.7229x; 2.7229x over previous
import jax
import jax.numpy as jnp
from jax import lax
from jax.experimental import pallas as pl
from jax.experimental.pallas import tpu as pltpu
from jax.experimental.pallas import tpu_sc as plsc

_NC = 2
_NS = 16
_L = 16
_D = 128
_CH = 128
_RG = 64


def _mesh():
    return plsc.VectorSubcoreMesh(
        core_axis_name="c", subcore_axis_name="s",
        num_cores=_NC, num_subcores=_NS)


def _make_conv_kernel(n_big, e_big, n_k, e_k):
    et = e_big // _NS
    n_chunks = et // _CH
    rpt = n_big // _NS
    et_k = e_k // _NS
    nk_chunks = et_k // _CH
    rpt_k = n_k // _NS

    out_type = [
        jax.ShapeDtypeStruct((n_big, _D), jnp.float32),
        jax.ShapeDtypeStruct((n_big, _D), jnp.float32),
        jax.ShapeDtypeStruct((n_big, _D), jnp.float32),
        jax.ShapeDtypeStruct((n_big, _D), jnp.float32),
        jax.ShapeDtypeStruct((n_k, _D), jnp.float32),
        jax.ShapeDtypeStruct((n_k, _D), jnp.float32),
    ]

    scratch = [
        pltpu.VMEM_SHARED((n_big, _D), jnp.float32),
        pltpu.VMEM((_CH,), jnp.int32),
        pltpu.VMEM((_CH,), jnp.int32),
        pltpu.VMEM((_CH,), jnp.float32),
        pltpu.VMEM((_CH, _D), jnp.float32),
        pltpu.VMEM((_RG, _D), jnp.float32),
        pltpu.VMEM((_RG, _D), jnp.float32),
        pltpu.VMEM((_RG, _D), jnp.float32),
        pltpu.SemaphoreType.DMA,
    ]

    def body(x0_u, src_u, dst_u, w_u, x0_q, src_q, dst_q, w_q,
             x0_k, src_k, dst_k, w_k,
             mean_u, x1_u, mean_q, x1_q, mean_k, x1_k,
             acc, srcv, dstv, wv, rows_v, a_v, b_v, z_v, sem):
        c = lax.axis_index("c")
        s = lax.axis_index("s")

        def edge_phase(x_hbm, src_h, dst_h, w_h, nch, epertile):
            def chunk(ci, _):
                off = s * epertile + ci * _CH
                pltpu.sync_copy(src_h.at[pl.ds(off, _CH)], srcv)
                pltpu.sync_copy(dst_h.at[pl.ds(off, _CH)], dstv)
                pltpu.sync_copy(w_h.at[pl.ds(off, _CH)], wv)
                pltpu.async_copy(x_hbm.at[srcv], rows_v, sem).wait()

                def scale_g(g, _):
                    w16 = wv[pl.ds(g * _L, _L)]
                    for e in range(_L):
                        r = g * _L + e
                        for q in range(_D // _L):
                            sl = pl.ds(q * _L, _L)
                            rows_v[r, sl] = rows_v[r, sl] * w16[e]
                    return 0

                lax.fori_loop(0, _CH // _L, scale_g, 0)
                pltpu.sync_copy(rows_v, acc.at[dstv], add=True)
                return 0

            lax.fori_loop(0, nch, chunk, 0)

        def combine1(x0_h, x1_h, row0, rows):
            rg = min(rows, _RG)
            av = a_v.at[pl.ds(0, rg)]
            bv = b_v.at[pl.ds(0, rg)]

            def cbody(ci, _):
                r0 = row0 + ci * rg
                pltpu.sync_copy(acc.at[pl.ds(r0, rg)], av)
                pltpu.sync_copy(x0_h.at[pl.ds(r0, rg)], bv)

                def upd(r, _):
                    for q in range(_D // _L):
                        sl = pl.ds(q * _L, _L)
                        a_v[r, sl] = a_v[r, sl] + 0.8 * b_v[r, sl]
                    return 0

                lax.fori_loop(0, rg, upd, 0, unroll=4)
                pltpu.sync_copy(av, x1_h.at[pl.ds(r0, rg)])
                pltpu.sync_copy(z_v.at[pl.ds(0, rg)], acc.at[pl.ds(r0, rg)])
                return 0

            lax.fori_loop(0, rows // rg, cbody, 0)

        def combine2(x0_h, x1_h, mean_h, row0, rows):
            rg = min(rows, _RG)
            av = a_v.at[pl.ds(0, rg)]
            bv = b_v.at[pl.ds(0, rg)]

            def cbody(ci, _):
                r0 = row0 + ci * rg
                pltpu.sync_copy(acc.at[pl.ds(r0, rg)], av)
                pltpu.sync_copy(x1_h.at[pl.ds(r0, rg)], bv)

                def upd1(r, _):
                    for q in range(_D // _L):
                        sl = pl.ds(q * _L, _L)
                        a_v[r, sl] = a_v[r, sl] + 1.8 * b_v[r, sl]
                    return 0

                lax.fori_loop(0, rg, upd1, 0, unroll=4)
                pltpu.sync_copy(x0_h.at[pl.ds(r0, rg)], bv)

                def upd2(r, _):
                    for q in range(_D // _L):
                        sl = pl.ds(q * _L, _L)
                        a_v[r, sl] = (a_v[r, sl] + b_v[r, sl]) * (1.0 / 3.0)
                    return 0

                lax.fori_loop(0, rg, upd2, 0, unroll=4)
                pltpu.sync_copy(av, mean_h.at[pl.ds(r0, rg)])
                pltpu.sync_copy(z_v.at[pl.ds(0, rg)], acc.at[pl.ds(r0, rg)])
                return 0

            lax.fori_loop(0, rows // rg, cbody, 0)

        is_u = c == 0
        is_q = c == 1

        z = jnp.zeros((_L,), jnp.float32)

        def zbody(r, _):
            for q in range(_D // _L):
                z_v[r, pl.ds(q * _L, _L)] = z
            return 0

        lax.fori_loop(0, _RG, zbody, 0)

        def z0(ci, _):
            pltpu.sync_copy(z_v, acc.at[pl.ds(s * rpt + ci * _RG, _RG)])
            return 0

        lax.fori_loop(0, rpt // _RG, z0, 0)
        plsc.subcore_barrier()

        @pl.when(is_u)
        def _():
            edge_phase(x0_u, src_u, dst_u, w_u, n_chunks, et)

        @pl.when(is_q)
        def _():
            edge_phase(x0_q, src_q, dst_q, w_q, n_chunks, et)
        plsc.subcore_barrier()

        @pl.when(is_u)
        def _():
            combine1(x0_u, x1_u, s * rpt, rpt)

        @pl.when(is_q)
        def _():
            combine1(x0_q, x1_q, s * rpt, rpt)
        plsc.subcore_barrier()

        @pl.when(is_u)
        def _():
            edge_phase(x1_u, src_u, dst_u, w_u, n_chunks, et)

        @pl.when(is_q)
        def _():
            edge_phase(x1_q, src_q, dst_q, w_q, n_chunks, et)
        plsc.subcore_barrier()

        @pl.when(is_u)
        def _():
            combine2(x0_u, x1_u, mean_u, s * rpt, rpt)

        @pl.when(is_q)
        def _():
            combine2(x0_q, x1_q, mean_q, s * rpt, rpt)
        plsc.subcore_barrier()

        @pl.when(is_u)
        def _():
            edge_phase(x0_k, src_k, dst_k, w_k, nk_chunks, et_k)
        plsc.subcore_barrier()

        @pl.when(is_u)
        def _():
            combine1(x0_k, x1_k, s * rpt_k, rpt_k)
        plsc.subcore_barrier()

        @pl.when(is_u)
        def _():
            edge_phase(x1_k, src_k, dst_k, w_k, nk_chunks, et_k)
        plsc.subcore_barrier()

        @pl.when(is_u)
        def _():
            combine2(x0_k, x1_k, mean_k, s * rpt_k, rpt_k)

    return pl.kernel(body, out_type=out_type, mesh=_mesh(),
                     scratch_types=scratch)


def _make_gather_kernel(n_big, n_k_unused, batch, q_cols):
    _QCH = 64

    out_type = [
        jax.ShapeDtypeStruct((batch, _D), jnp.float32),
        jax.ShapeDtypeStruct((batch, _D), jnp.float32),
        jax.ShapeDtypeStruct((batch, q_cols), jnp.float32),
    ]
    scratch = [
        pltpu.VMEM((_CH,), jnp.int32),
        pltpu.VMEM((_CH, _D), jnp.float32),
        pltpu.VMEM((_QCH,), jnp.int32),
        pltpu.VMEM((_QCH, q_cols), jnp.float32),
        pltpu.SemaphoreType.DMA,
    ]

    def body(mean_u, mean_q, uid, qid, qt, bs_o, be_o, qb_o,
             idv, bb_v, qidv, q_v, sem):
        c = lax.axis_index("c")
        s = lax.axis_index("s")
        w = c * _NS + s
        per_w = batch // (_NC * _NS)

        def emb_gather(ids_h, table_h, out_h):
            for j in range(per_w // _CH):
                b0 = w * per_w + j * _CH
                pltpu.sync_copy(ids_h.at[pl.ds(b0, _CH)], idv)
                pltpu.async_copy(table_h.at[idv], bb_v, sem).wait()
                pltpu.sync_copy(bb_v, out_h.at[pl.ds(b0, _CH)])

        emb_gather(uid, mean_u, bs_o)
        emb_gather(qid, mean_q, be_o)
        for j in range(per_w // _QCH):
            b0 = w * per_w + j * _QCH
            pltpu.sync_copy(qid.at[pl.ds(b0, _QCH)], qidv)
            pltpu.async_copy(qt.at[qidv], q_v, sem).wait()
            pltpu.sync_copy(q_v, qb_o.at[pl.ds(b0, _QCH)])

    return pl.kernel(body, out_type=out_type, mesh=_mesh(),
                     scratch_types=scratch)


def _pad_edges(edge_index, w, e_pad):
    e = edge_index.shape[1]
    src = jnp.pad(edge_index[1], (0, e_pad - e))
    dst = jnp.pad(edge_index[0], (0, e_pad - e))
    wp = jnp.pad(w, (0, e_pad - e))
    return src, dst, wp


def _leaky(x):
    return jnp.where(x >= 0, x, 0.8 * x)


def _dense_body(bs, be, qb, mk,
                ws, bsb, we, beb, wk, bkb, wdr, bdr,
                w1, b1, w2, b2, w3, b3, w4r, b4r, out_ref):
    f32 = jnp.float32
    dot = lambda a, b: lax.dot_general(a, b, (((1,), (0,)), ((), ())),
                                       preferred_element_type=f32)
    dot_t = lambda a, b: lax.dot_general(a, b, (((1,), (1,)), ((), ())),
                                         preferred_element_type=f32)
    sf = _leaky(dot(bs[...], ws[...]) + bsb[...])
    ef = _leaky(dot(be[...], we[...]) + beb[...])
    kf = _leaky(dot(mk[...], wk[...]) + bkb[...])
    disc = jax.nn.sigmoid(
        jnp.sum(be[...] * wdr[...], axis=1, keepdims=True) + bdr[...])
    state = disc * dot_t(sf - ef, kf) * qb[...]
    h = jnp.tanh(dot(state, w1[...]) + b1[...])
    h = jnp.tanh(dot(h, w2[...]) + b2[...])
    h = jnp.tanh(dot(h, w3[...]) + b3[...])
    p = jax.nn.sigmoid(
        jnp.sum(h * w4r[...], axis=1, keepdims=True) + b4r[...])
    out_ref[...] = p[:, 0]


def kernel(user_id, question_id, q_table, user_edge_index, user_edge_weight,
           question_edge_index, question_edge_weight, concept_edge_index,
           concept_edge_weight, student_emb, exercise_emb, knowledge_emb,
           W_s, b_s, W_e, b_e, W_k, b_k, W_d, b_d,
           W1, b1, W2, b2, W3, b3, W4, b4):
    B = user_id.shape[0]
    n_raw = student_emb.shape[0]
    n_k = knowledge_emb.shape[0]
    q_cols = q_table.shape[1]

    def pad_to(e, step=_NS * _CH):
        return ((e + step - 1) // step) * step

    n_big = pad_to(n_raw, _NS * _RG)
    e_big = pad_to(user_edge_index.shape[1])
    e_k = pad_to(concept_edge_index.shape[1])

    conv = _make_conv_kernel(n_big, e_big, n_k, e_k)
    gath = _make_gather_kernel(n_big, n_k, B, q_cols)

    su, du, wu = _pad_edges(user_edge_index, user_edge_weight, e_big)
    sq, dq, wq = _pad_edges(question_edge_index, question_edge_weight, e_big)
    sk, dk, wk = _pad_edges(concept_edge_index, concept_edge_weight, e_k)

    node_pad = ((0, n_big - n_raw), (0, 0))
    mean_u, _, mean_q, _, mk, _ = conv(
        jnp.pad(student_emb, node_pad), su, du, wu,
        jnp.pad(exercise_emb, node_pad), sq, dq, wq,
        knowledge_emb, sk, dk, wk)

    bs, be, qb = gath(mean_u, mean_q, user_id.astype(jnp.int32),
                      question_id.astype(jnp.int32), q_table)

    TB = 512
    grid = (B // TB,)
    bspec = pl.BlockSpec((TB, _D), lambda i: (i, 0))
    qspec = pl.BlockSpec((TB, q_cols), lambda i: (i, 0))
    const = lambda shape: pl.BlockSpec(shape, lambda i: (0, 0))

    operands = [
        bs, be, qb, mk,
        W_s, b_s.reshape(1, -1), W_e, b_e.reshape(1, -1),
        W_k, b_k.reshape(1, -1), W_d.reshape(1, _D), b_d.reshape(1, 1),
        W1, b1.reshape(1, -1), W2, b2.reshape(1, -1), W3, b3.reshape(1, -1),
        W4.reshape(1, -1), b4.reshape(1, 1),
    ]
    in_specs = [bspec, bspec, qspec, const((n_k, _D))]
    for op in operands[4:]:
        in_specs.append(const(op.shape))

    predict = pl.pallas_call(
        _dense_body,
        grid=grid,
        in_specs=in_specs,
        out_specs=pl.BlockSpec((TB,), lambda i: (i,)),
        out_shape=jax.ShapeDtypeStruct((B,), jnp.float32),
    )(*operands)
    return predict


# speedup vs baseline: 4.0860x; 1.5006x over previous
import jax
import jax.numpy as jnp
from jax import lax
from jax.experimental import pallas as pl
from jax.experimental.pallas import tpu as pltpu
from jax.experimental.pallas import tpu_sc as plsc

_NC = 2
_NS = 16
_L = 16
_D = 128
_CH = 128
_RG = 32


def _mesh():
    return plsc.VectorSubcoreMesh(
        core_axis_name="c", subcore_axis_name="s",
        num_cores=_NC, num_subcores=_NS)


def _make_conv_kernel(n_big, e_big, n_k, e_k):
    et = e_big // _NS
    n_chunks = et // _CH
    rpt = n_big // _NS
    et_k = e_k // _NS
    nk_chunks = et_k // _CH
    rpt_k = n_k // _NS

    out_type = [
        jax.ShapeDtypeStruct((n_big, _D), jnp.float32),
        jax.ShapeDtypeStruct((n_big, _D), jnp.float32),
        jax.ShapeDtypeStruct((n_big, _D), jnp.float32),
        jax.ShapeDtypeStruct((n_big, _D), jnp.float32),
        jax.ShapeDtypeStruct((n_k, _D), jnp.float32),
        jax.ShapeDtypeStruct((n_k, _D), jnp.float32),
    ]

    scratch = [
        pltpu.VMEM_SHARED((n_big, _D), jnp.float32),
        pltpu.VMEM((3, _CH), jnp.int32),
        pltpu.VMEM((3, _CH), jnp.int32),
        pltpu.VMEM((_CH,), jnp.int32),
        pltpu.VMEM((_CH,), jnp.int32),
        pltpu.VMEM((_CH, _D), jnp.float32),
        pltpu.VMEM((_CH, _D), jnp.float32),
        pltpu.VMEM((_RG, _D), jnp.float32),
        pltpu.VMEM((_RG, _D), jnp.float32),
        pltpu.VMEM((_RG, _D), jnp.float32),
        pltpu.SemaphoreType.DMA,
        pltpu.SemaphoreType.DMA,
        pltpu.SemaphoreType.DMA,
        pltpu.SemaphoreType.DMA,
        pltpu.SemaphoreType.DMA,
        pltpu.SemaphoreType.DMA,
    ]

    def body(x0_u, pk_u, x0_q, pk_q, x0_k, pk_k,
             mean_u, x1_u, mean_q, x1_q, mean_k, x1_k,
             acc, pv0, pv1, dv0, dv1, rows0, rows1, a_v, b_v, z_v,
             is0, is1, gs0, gs1, ss0, ss1):
        c = lax.axis_index("c")
        s = lax.axis_index("s")

        def scale(rv, pv):
            def scale_g(g, _):
                w16 = plsc.bitcast(pv[2, pl.ds(g * _L, _L)], jnp.float32)
                for e in range(_L):
                    r = g * _L + e
                    for q in range(_D // _L):
                        sl = pl.ds(q * _L, _L)
                        rv[r, sl] = rv[r, sl] * w16[e]
                return 0

            lax.fori_loop(0, _CH // _L, scale_g, 0)

        def cpdst(pv, dv):
            for g in range(_CH // _L):
                sl = pl.ds(g * _L, _L)
                dv[sl] = pv[1, sl]

        def edge_pipe(x_hbm, pk_h, epertile, nchunks):
            npairs = nchunks // 2

            def sidx(ci, pv, sm):
                off = s * epertile + ci * _CH
                pltpu.async_copy(pk_h.at[:, pl.ds(off, _CH)], pv, sm)

            def widx(pv, sm):
                pltpu.make_async_copy(
                    pk_h.at[:, pl.ds(0, _CH)], pv, sm).wait()

            def sgat(pv, rv, sm):
                pltpu.async_copy(x_hbm.at[pv.at[0]], rv, sm)

            def wgat(rv, sm):
                pltpu.make_async_copy(
                    x_hbm.at[pl.ds(0, _CH)], rv, sm).wait()

            def sscat(rv, dv, sm):
                pltpu.async_copy(rv, acc.at[dv], sm, add=True)

            def wscat(rv, dv, sm):
                pltpu.make_async_copy(rv, acc.at[dv], sm).wait()

            sidx(0, pv0, is0)
            widx(pv0, is0)
            sgat(pv0, rows0, gs0)
            sidx(1, pv1, is1)

            def pair(i, _):
                @pl.when(i > 0)
                def _():
                    wscat(rows1, dv1, ss1)
                widx(pv1, is1)
                sgat(pv1, rows1, gs1)

                wgat(rows0, gs0)
                scale(rows0, pv0)
                cpdst(pv0, dv0)
                sscat(rows0, dv0, ss0)

                @pl.when(i < npairs - 1)
                def _():
                    sidx(2 * i + 2, pv0, is0)

                wgat(rows1, gs1)
                scale(rows1, pv1)
                cpdst(pv1, dv1)
                sscat(rows1, dv1, ss1)

                @pl.when(i < npairs - 1)
                def _():
                    wscat(rows0, dv0, ss0)
                    widx(pv0, is0)
                    sgat(pv0, rows0, gs0)
                    sidx(2 * i + 3, pv1, is1)
                return 0

            lax.fori_loop(0, npairs, pair, 0)
            wscat(rows0, dv0, ss0)
            wscat(rows1, dv1, ss1)

        def edge_phase(x_hbm, pk_h, nch, epertile):
            def chunk(ci, _):
                off = s * epertile + ci * _CH
                pltpu.sync_copy(pk_h.at[:, pl.ds(off, _CH)], pv0)
                pltpu.async_copy(x_hbm.at[pv0.at[0]], rows0, gs0).wait()
                scale(rows0, pv0)
                pltpu.sync_copy(rows0, acc.at[pv0.at[1]], add=True)
                return 0

            lax.fori_loop(0, nch, chunk, 0)

        def combine1(x0_h, x1_h, row0, rows):
            rg = min(rows, _RG)
            av = a_v.at[pl.ds(0, rg)]
            bv = b_v.at[pl.ds(0, rg)]

            def cbody(ci, _):
                r0 = row0 + ci * rg
                pltpu.sync_copy(acc.at[pl.ds(r0, rg)], av)
                pltpu.sync_copy(x0_h.at[pl.ds(r0, rg)], bv)

                def upd(r, _):
                    for q in range(_D // _L):
                        sl = pl.ds(q * _L, _L)
                        a_v[r, sl] = a_v[r, sl] + 0.8 * b_v[r, sl]
                    return 0

                lax.fori_loop(0, rg, upd, 0, unroll=4)
                pltpu.sync_copy(av, x1_h.at[pl.ds(r0, rg)])
                pltpu.sync_copy(z_v.at[pl.ds(0, rg)], acc.at[pl.ds(r0, rg)])
                return 0

            lax.fori_loop(0, rows // rg, cbody, 0)

        def combine2(x0_h, x1_h, mean_h, row0, rows):
            rg = min(rows, _RG)
            av = a_v.at[pl.ds(0, rg)]
            bv = b_v.at[pl.ds(0, rg)]

            def cbody(ci, _):
                r0 = row0 + ci * rg
                pltpu.sync_copy(acc.at[pl.ds(r0, rg)], av)
                pltpu.sync_copy(x1_h.at[pl.ds(r0, rg)], bv)

                def upd1(r, _):
                    for q in range(_D // _L):
                        sl = pl.ds(q * _L, _L)
                        a_v[r, sl] = a_v[r, sl] + 1.8 * b_v[r, sl]
                    return 0

                lax.fori_loop(0, rg, upd1, 0, unroll=4)
                pltpu.sync_copy(x0_h.at[pl.ds(r0, rg)], bv)

                def upd2(r, _):
                    for q in range(_D // _L):
                        sl = pl.ds(q * _L, _L)
                        a_v[r, sl] = (a_v[r, sl] + b_v[r, sl]) * (1.0 / 3.0)
                    return 0

                lax.fori_loop(0, rg, upd2, 0, unroll=4)
                pltpu.sync_copy(av, mean_h.at[pl.ds(r0, rg)])
                pltpu.sync_copy(z_v.at[pl.ds(0, rg)], acc.at[pl.ds(r0, rg)])
                return 0

            lax.fori_loop(0, rows // rg, cbody, 0)

        is_u = c == 0
        is_q = c == 1

        z = jnp.zeros((_L,), jnp.float32)

        def zbody(r, _):
            for q in range(_D // _L):
                z_v[r, pl.ds(q * _L, _L)] = z
            return 0

        lax.fori_loop(0, _RG, zbody, 0)

        def z0(ci, _):
            pltpu.sync_copy(z_v, acc.at[pl.ds(s * rpt + ci * _RG, _RG)])
            return 0

        lax.fori_loop(0, rpt // _RG, z0, 0)
        plsc.subcore_barrier()

        @pl.when(is_u)
        def _():
            edge_pipe(x0_u, pk_u, et, n_chunks)

        @pl.when(is_q)
        def _():
            edge_pipe(x0_q, pk_q, et, n_chunks)
        plsc.subcore_barrier()

        @pl.when(is_u)
        def _():
            combine1(x0_u, x1_u, s * rpt, rpt)

        @pl.when(is_q)
        def _():
            combine1(x0_q, x1_q, s * rpt, rpt)
        plsc.subcore_barrier()

        @pl.when(is_u)
        def _():
            edge_pipe(x1_u, pk_u, et, n_chunks)

        @pl.when(is_q)
        def _():
            edge_pipe(x1_q, pk_q, et, n_chunks)
        plsc.subcore_barrier()

        @pl.when(is_u)
        def _():
            combine2(x0_u, x1_u, mean_u, s * rpt, rpt)

        @pl.when(is_q)
        def _():
            combine2(x0_q, x1_q, mean_q, s * rpt, rpt)
        plsc.subcore_barrier()

        @pl.when(is_u)
        def _():
            edge_phase(x0_k, pk_k, nk_chunks, et_k)
        plsc.subcore_barrier()

        @pl.when(is_u)
        def _():
            combine1(x0_k, x1_k, s * rpt_k, rpt_k)
        plsc.subcore_barrier()

        @pl.when(is_u)
        def _():
            edge_phase(x1_k, pk_k, nk_chunks, et_k)
        plsc.subcore_barrier()

        @pl.when(is_u)
        def _():
            combine2(x0_k, x1_k, mean_k, s * rpt_k, rpt_k)

    return pl.kernel(body, out_type=out_type, mesh=_mesh(),
                     scratch_types=scratch,
                     compiler_params=pltpu.CompilerParams(
                         needs_layout_passes=False))


def _make_gather_kernel(n_big, n_k_unused, batch, q_cols):
    _QCH = 64

    out_type = [
        jax.ShapeDtypeStruct((batch, _D), jnp.float32),
        jax.ShapeDtypeStruct((batch, _D), jnp.float32),
        jax.ShapeDtypeStruct((batch, q_cols), jnp.float32),
    ]
    scratch = [
        pltpu.VMEM((_CH,), jnp.int32),
        pltpu.VMEM((_CH, _D), jnp.float32),
        pltpu.VMEM((_QCH,), jnp.int32),
        pltpu.VMEM((_QCH, q_cols), jnp.float32),
        pltpu.SemaphoreType.DMA,
    ]

    def body(mean_u, mean_q, uid, qid, qt, bs_o, be_o, qb_o,
             idv, bb_v, qidv, q_v, sem):
        c = lax.axis_index("c")
        s = lax.axis_index("s")
        w = c * _NS + s
        per_w = batch // (_NC * _NS)

        def emb_gather(ids_h, table_h, out_h):
            for j in range(per_w // _CH):
                b0 = w * per_w + j * _CH
                pltpu.sync_copy(ids_h.at[pl.ds(b0, _CH)], idv)
                pltpu.async_copy(table_h.at[idv], bb_v, sem).wait()
                pltpu.sync_copy(bb_v, out_h.at[pl.ds(b0, _CH)])

        emb_gather(uid, mean_u, bs_o)
        emb_gather(qid, mean_q, be_o)
        for j in range(per_w // _QCH):
            b0 = w * per_w + j * _QCH
            pltpu.sync_copy(qid.at[pl.ds(b0, _QCH)], qidv)
            pltpu.async_copy(qt.at[qidv], q_v, sem).wait()
            pltpu.sync_copy(q_v, qb_o.at[pl.ds(b0, _QCH)])

    return pl.kernel(body, out_type=out_type, mesh=_mesh(),
                     scratch_types=scratch)


def _pack_edges(edge_index, w, e_pad):
    e = edge_index.shape[1]
    pad = (0, e_pad - e)
    return jnp.stack([
        jnp.pad(edge_index[1], pad).astype(jnp.int32),
        jnp.pad(edge_index[0], pad).astype(jnp.int32),
        lax.bitcast_convert_type(jnp.pad(w, pad), jnp.int32),
    ])


def _leaky(x):
    return jnp.where(x >= 0, x, 0.8 * x)


def _dense_body(bs, be, qb, mk,
                ws, bsb, we, beb, wk, bkb, wdr, bdr,
                w1, b1, w2, b2, w3, b3, w4r, b4r, out_ref):
    f32 = jnp.float32
    dot = lambda a, b: lax.dot_general(a, b, (((1,), (0,)), ((), ())),
                                       preferred_element_type=f32)
    dot_t = lambda a, b: lax.dot_general(a, b, (((1,), (1,)), ((), ())),
                                         preferred_element_type=f32)
    sf = _leaky(dot(bs[...], ws[...]) + bsb[...])
    ef = _leaky(dot(be[...], we[...]) + beb[...])
    kf = _leaky(dot(mk[...], wk[...]) + bkb[...])
    disc = jax.nn.sigmoid(
        jnp.sum(be[...] * wdr[...], axis=1, keepdims=True) + bdr[...])
    state = disc * dot_t(sf - ef, kf) * qb[...]
    h = jnp.tanh(dot(state, w1[...]) + b1[...])
    h = jnp.tanh(dot(h, w2[...]) + b2[...])
    h = jnp.tanh(dot(h, w3[...]) + b3[...])
    p = jax.nn.sigmoid(
        jnp.sum(h * w4r[...], axis=1, keepdims=True) + b4r[...])
    out_ref[...] = p[:, 0]


def kernel(user_id, question_id, q_table, user_edge_index, user_edge_weight,
           question_edge_index, question_edge_weight, concept_edge_index,
           concept_edge_weight, student_emb, exercise_emb, knowledge_emb,
           W_s, b_s, W_e, b_e, W_k, b_k, W_d, b_d,
           W1, b1, W2, b2, W3, b3, W4, b4):
    B = user_id.shape[0]
    n_raw = student_emb.shape[0]
    n_k = knowledge_emb.shape[0]
    q_cols = q_table.shape[1]

    def pad_to(e, step=2 * _NS * _CH):
        return ((e + step - 1) // step) * step

    n_big = pad_to(n_raw, _NS * _CH)
    e_big = pad_to(user_edge_index.shape[1])
    e_k = pad_to(concept_edge_index.shape[1], _NS * _CH)

    conv = _make_conv_kernel(n_big, e_big, n_k, e_k)
    gath = _make_gather_kernel(n_big, n_k, B, q_cols)

    pk_u = _pack_edges(user_edge_index, user_edge_weight, e_big)
    pk_q = _pack_edges(question_edge_index, question_edge_weight, e_big)
    pk_k = _pack_edges(concept_edge_index, concept_edge_weight, e_k)

    node_pad = ((0, n_big - n_raw), (0, 0))
    mean_u, _, mean_q, _, mk, _ = conv(
        jnp.pad(student_emb, node_pad), pk_u,
        jnp.pad(exercise_emb, node_pad), pk_q,
        knowledge_emb, pk_k)

    bs, be, qb = gath(mean_u, mean_q, user_id.astype(jnp.int32),
                      question_id.astype(jnp.int32), q_table)

    TB = 512
    grid = (B // TB,)
    bspec = pl.BlockSpec((TB, _D), lambda i: (i, 0))
    qspec = pl.BlockSpec((TB, q_cols), lambda i: (i, 0))
    const = lambda shape: pl.BlockSpec(shape, lambda i: (0, 0))

    operands = [
        bs, be, qb, mk,
        W_s, b_s.reshape(1, -1), W_e, b_e.reshape(1, -1),
        W_k, b_k.reshape(1, -1), W_d.reshape(1, _D), b_d.reshape(1, 1),
        W1, b1.reshape(1, -1), W2, b2.reshape(1, -1), W3, b3.reshape(1, -1),
        W4.reshape(1, -1), b4.reshape(1, 1),
    ]
    in_specs = [bspec, bspec, qspec, const((n_k, _D))]
    for op in operands[4:]:
        in_specs.append(const(op.shape))

    predict = pl.pallas_call(
        _dense_body,
        grid=grid,
        in_specs=in_specs,
        out_specs=pl.BlockSpec((TB,), lambda i: (i,)),
        out_shape=jax.ShapeDtypeStruct((B,), jnp.float32),
    )(*operands)
    return predict
